# Initial kernel scaffold; baseline (speedup 1.0000x reference)
import jax
import jax.numpy as jnp
from jax.experimental import pallas as pl


def kernel(values):
    raise NotImplementedError("write your pallas kernel here")


# speedup vs baseline: 2.0260x; 2.0260x over previous
import jax
import jax.numpy as jnp
from jax import lax
from jax.experimental import pallas as pl

_COORDS = ((0, 2, 3), (1, 1, 2), (2, 1, 4), (3, 5, 1))


def _body(x_ref, o_ref):
    x = x_ref[...]
    i = lax.broadcasted_iota(jnp.int32, x.shape, 0)
    j = lax.broadcasted_iota(jnp.int32, x.shape, 1)
    k = lax.broadcasted_iota(jnp.int32, x.shape, 2)
    mask = None
    for (a, b, c) in _COORDS:
        m = (i == a) & (j == b) & (k == c)
        mask = m if mask is None else (mask | m)
    o_ref[...] = jnp.sum(jnp.where(mask, x, 0.0)).reshape(1, 1)


def kernel(values):
    out = pl.pallas_call(
        _body,
        out_shape=jax.ShapeDtypeStruct((1, 1), jnp.float32),
        grid=(1,),
        in_specs=[pl.BlockSpec((4, 8, 128), lambda i: (0, 0, 0))],
        out_specs=pl.BlockSpec((1, 1), lambda i: (0, 0)),
    )(values)
    return out[0, 0]
